# baseline (speedup 1.0000x reference)
import functools

import jax
import jax.numpy as jnp
from jax import lax
from jax.experimental import pallas as pl
from jax.experimental.pallas import tpu as pltpu
from jax.experimental.pallas import tpu_sc as plsc

VOCAB = 1000000
EMBED = 64
BATCH = 16384
CTX = 5

NC = 2
NS = 16
L = 16
NW = NC * NS
BPW = BATCH // NW
CHUNK = 128
NCHUNK = BPW // CHUNK
EV = EMBED // L

TBLK = 8192


def _transpose_body(tt_ref, out_ref):
    out_ref[...] = tt_ref[...].T


def _retile(table_t):
    grid = (VOCAB + TBLK - 1) // TBLK
    return pl.pallas_call(
        _transpose_body,
        grid=(grid,),
        in_specs=[pl.BlockSpec((EMBED, TBLK), lambda j: (0, j))],
        out_specs=pl.BlockSpec((TBLK, EMBED), lambda j: (j, 0)),
        out_shape=jax.ShapeDtypeStruct((VOCAB, EMBED), jnp.float32),
    )(table_t)


def _body(tgt_hbm, ctx_hbm, ttab_hbm, ctab_hbm, out_hbm,
          idx_t, idx_c, wbuf, cbuf, obuf, sem0, sem1, sem_out):
    wid = lax.axis_index("s") * NC + lax.axis_index("c")
    base = wid * BPW
    sems = (sem0, sem1)

    def fire(k, s):
        cb = base + k * CHUNK
        pltpu.sync_copy(tgt_hbm.at[pl.ds(cb, CHUNK)], idx_t.at[s])
        pltpu.sync_copy(ctx_hbm.at[pl.ds(cb * CTX, CHUNK * CTX)], idx_c.at[s])
        copies = [pltpu.async_copy(ttab_hbm.at[idx_t.at[s]], wbuf.at[s],
                                   sems[s])]
        for g in range(CTX):
            copies.append(pltpu.async_copy(
                ctab_hbm.at[idx_c.at[s].at[pl.ds(g * CHUNK, CHUNK)]],
                cbuf.at[s].at[pl.ds(g * CHUNK, CHUNK)], sems[s]))
        return copies

    zero = jnp.zeros((L,), jnp.float32)
    out_copies = [None, None]
    pending = fire(0, 0)
    for k in range(NCHUNK):
        s = k % 2
        next_pending = fire(k + 1, 1 - s) if k + 1 < NCHUNK else None
        if out_copies[s] is not None:
            out_copies[s].wait()
        oslot = obuf.at[s]
        for i in range(CHUNK * CTX // L):
            oslot[pl.ds(i * L, L)] = zero
        for c in pending:
            c.wait()

        wslot = wbuf.at[s]
        cslot = cbuf.at[s]

        @plsc.parallel_loop(0, CHUNK)
        def b_body(b):
            w = [wslot[b, pl.ds(j * L, L)] for j in range(EV)]
            for c in range(CTX):
                r = b * CTX + c
                acc = w[0] * cslot[r, pl.ds(0, L)]
                for j in range(1, EV):
                    acc = acc + w[j] * cslot[r, pl.ds(j * L, L)]
                plsc.addupdate_scatter(oslot, [jnp.full((L,), r, jnp.int32)],
                                       acc)

        cb = base + k * CHUNK
        out_copies[s] = pltpu.async_copy(
            oslot, out_hbm.at[pl.ds(cb * CTX, CHUNK * CTX)], sem_out)
        pending = next_pending
    for oc in out_copies:
        if oc is not None:
            oc.wait()


@jax.jit
def kernel(target, context, target_table, context_table):
    mesh = plsc.VectorSubcoreMesh(core_axis_name="c", subcore_axis_name="s",
                                  num_cores=NC, num_subcores=NS)
    ctx_flat = context.reshape(BATCH * CTX)
    target_table = _retile(target_table.T)
    context_table = _retile(context_table.T)
    run = functools.partial(
        pl.kernel,
        out_type=jax.ShapeDtypeStruct((BATCH * CTX,), jnp.float32),
        mesh=mesh,
        scratch_types=[
            pltpu.VMEM((2, CHUNK), jnp.int32),
            pltpu.VMEM((2, CHUNK * CTX), jnp.int32),
            pltpu.VMEM((2, CHUNK, EMBED), jnp.float32),
            pltpu.VMEM((2, CHUNK * CTX, EMBED), jnp.float32),
            pltpu.VMEM((2, CHUNK * CTX), jnp.float32),
            pltpu.SemaphoreType.DMA,
            pltpu.SemaphoreType.DMA,
            pltpu.SemaphoreType.DMA,
        ],
        compiler_params=pltpu.CompilerParams(needs_layout_passes=False,
                                             use_tc_tiling_on_sc=False),
    )(_body)
    out = run(target, ctx_flat, target_table, context_table)
    return out.reshape(BATCH, CTX)
